# baseline (speedup 1.0000x reference)
import functools

import jax
import jax.numpy as jnp
from jax import lax
from jax.experimental import pallas as pl
from jax.experimental.pallas import tpu as pltpu
from jax.experimental.pallas import tpu_sc as plsc

N = 16384
F = 8
NUM_TYPES = 8
EMBED_DIM = 256

_INFO = plsc.get_sparse_core_info()
NC, NS, L = _INFO.num_cores, _INFO.num_subcores, _INFO.num_lanes
NW = NC * NS
B_PER_W = N // NW
CHUNK = 128
N_CHUNKS = B_PER_W // CHUNK


def _make_kernel():
  mesh = plsc.VectorSubcoreMesh(core_axis_name="c", subcore_axis_name="s")

  @functools.partial(
      pl.kernel,
      mesh=mesh,
      compiler_params=pltpu.CompilerParams(needs_layout_passes=False),
      out_type=jax.ShapeDtypeStruct((N, EMBED_DIM), jnp.float32),
      scratch_types=[
          pltpu.VMEM((B_PER_W * F,), jnp.float32),
          pltpu.VMEM((B_PER_W,), jnp.int32),
          pltpu.VMEM((CHUNK, EMBED_DIM), jnp.float32),
          pltpu.VMEM((CHUNK, EMBED_DIM), jnp.float32),
          pltpu.SemaphoreType.DMA,
          pltpu.SemaphoreType.DMA,
      ],
  )
  def k(inputs_flat_hbm, table_hbm, out_hbm, vals_v, idx_v, rows0, rows1,
        gsem, osem):
    s = lax.axis_index("s")
    wid = s * NC + lax.axis_index("c")
    base = wid * B_PER_W

    pltpu.sync_copy(inputs_flat_hbm.at[pl.ds(base * F, B_PER_W * F)], vals_v)
    lanes = lax.iota(jnp.int32, L)
    for i in range(B_PER_W // L):
      pos = lanes * F + (i * L * F + 2)
      col2 = plsc.load_gather(vals_v, [pos])
      idx_v[pl.ds(i * L, L)] = jnp.clip(col2.astype(jnp.int32), 0, NUM_TYPES - 1)

    bufs = (rows0, rows1)
    out_dma = [None, None]
    for t in range(N_CHUNKS):
      b = bufs[t % 2]
      if t >= 2:
        out_dma[t % 2].wait()
      pltpu.async_copy(
          table_hbm.at[idx_v.at[pl.ds(t * CHUNK, CHUNK)]], b, gsem
      ).wait()
      out_dma[t % 2] = pltpu.async_copy(
          b, out_hbm.at[pl.ds(base + t * CHUNK, CHUNK)], osem
      )
    out_dma[0].wait()
    out_dma[1].wait()

  return k


_kernel_call = _make_kernel()


@jax.jit
def kernel(inputs, type_embed):
  if inputs.ndim == 3:
    inputs = inputs[0]
  return _kernel_call(inputs.reshape(-1), type_embed)


# speedup vs baseline: 1.1956x; 1.1956x over previous
import functools

import jax
import jax.numpy as jnp
from jax import lax
from jax.experimental import pallas as pl
from jax.experimental.pallas import tpu as pltpu
from jax.experimental.pallas import tpu_sc as plsc

N = 16384
F = 8
NUM_TYPES = 8
EMBED_DIM = 256

_INFO = plsc.get_sparse_core_info()
NC, NS, L = _INFO.num_cores, _INFO.num_subcores, _INFO.num_lanes
NW = NC * NS
B_PER_W = N // NW
CHUNK = 128
N_CHUNKS = B_PER_W // CHUNK
GROUPS_PER_CHUNK = CHUNK // L


def _make_kernel():
  mesh = plsc.VectorSubcoreMesh(core_axis_name="c", subcore_axis_name="s")

  @functools.partial(
      pl.kernel,
      mesh=mesh,
      compiler_params=pltpu.CompilerParams(needs_layout_passes=False),
      out_type=jax.ShapeDtypeStruct((N * EMBED_DIM,), jnp.float32),
      scratch_types=[
          pltpu.VMEM((NUM_TYPES * EMBED_DIM,), jnp.float32),
          pltpu.VMEM((B_PER_W * F,), jnp.float32),
          pltpu.VMEM((B_PER_W,), jnp.int32),
          pltpu.VMEM((CHUNK * EMBED_DIM,), jnp.float32),
          pltpu.VMEM((CHUNK * EMBED_DIM,), jnp.float32),
          pltpu.SemaphoreType.DMA,
      ],
  )
  def k(inputs_flat_hbm, table_flat_hbm, out_hbm, table_v, vals_v, posb_v,
        buf0, buf1, osem):
    wid = lax.axis_index("s") * NC + lax.axis_index("c")
    base = wid * B_PER_W

    pltpu.sync_copy(table_flat_hbm, table_v)
    pltpu.sync_copy(inputs_flat_hbm.at[pl.ds(base * F, B_PER_W * F)], vals_v)

    lanes = lax.iota(jnp.int32, L)
    for i in range(B_PER_W // L):
      pos = lanes * F + (i * L * F + 2)
      col2 = plsc.load_gather(vals_v, [pos])
      idx = jnp.clip(col2.astype(jnp.int32), 0, NUM_TYPES - 1)
      posb_v[pl.ds(i * L, L)] = idx * EMBED_DIM

    outlane = lanes * EMBED_DIM
    bufs = (buf0, buf1)
    out_dma = [None, None]
    for t in range(N_CHUNKS):
      b = bufs[t % 2]
      if t >= 2:
        out_dma[t % 2].wait()
      for g in range(GROUPS_PER_CHUNK):
        posbase = posb_v[pl.ds((t * GROUPS_PER_CHUNK + g) * L, L)]
        outb = outlane + g * (L * EMBED_DIM)

        @plsc.parallel_loop(0, EMBED_DIM, unroll=8)
        def _(c, posbase=posbase, outb=outb, b=b):
          v = plsc.load_gather(table_v, [posbase + c])
          plsc.store_scatter(b, [outb + c], v)

      out_dma[t % 2] = pltpu.async_copy(
          b,
          out_hbm.at[pl.ds((base + t * CHUNK) * EMBED_DIM, CHUNK * EMBED_DIM)],
          osem,
      )
    out_dma[0].wait()
    out_dma[1].wait()

  return k


_kernel_call = _make_kernel()


@jax.jit
def kernel(inputs, type_embed):
  if inputs.ndim == 3:
    inputs = inputs[0]
  out_flat = _kernel_call(inputs.reshape(-1), type_embed.reshape(-1))
  return out_flat.reshape(N, EMBED_DIM)
